# baseline (speedup 1.0000x reference)
import functools

import jax
import jax.numpy as jnp
from jax import lax
from jax.experimental import pallas as pl
from jax.experimental.pallas import tpu as pltpu
from jax.experimental.pallas import tpu_sc as plsc

_TEMP = 0.05
_B = 1024
_D = 64
_N = 100000
_BN = 4000
_GRID = _N // _BN
_LOG2E = 1.4426950408889634
_LN2 = 0.6931471805599453
_C2 = 84.0

_NC = 2
_NS = 16
_NW = _NC * _NS
_BW = _B // _NW


def _lse_kernel(x_ref, f_ref, out_ref, fw_ref, mc_ref, s_ref):
    i = pl.program_id(0)

    @pl.when(i == 0)
    def _init():
        x2 = x_ref[...]
        m2 = jnp.sqrt(jnp.sum(x2 * x2, axis=1, keepdims=True))
        mc_ref[...] = m2 - _C2
        s_ref[...] = jnp.zeros_like(s_ref)

    f = f_ref[...]
    f3 = f.reshape(_BN // 2, 2, _D)
    fw_ref[...] = jnp.concatenate([f3[:, 0, :], f3[:, 1, :]], axis=1)

    z = jax.lax.dot_general(
        x_ref[...].astype(jnp.bfloat16), f.astype(jnp.bfloat16),
        (((1,), (1,)), ((), ())),
        preferred_element_type=jnp.float32,
    )
    e = jnp.exp2(z - mc_ref[...])
    s_ref[...] += jnp.sum(e, axis=1, keepdims=True)

    @pl.when(i == _GRID - 1)
    def _fin():
        out_ref[...] = mc_ref[...] + jnp.log2(s_ref[...])


_sc_mesh = plsc.VectorSubcoreMesh(core_axis_name="c", subcore_axis_name="s")


@functools.partial(
    pl.kernel,
    mesh=_sc_mesh,
    out_type=jax.ShapeDtypeStruct((_B, 2 * _D), jnp.float32),
    scratch_types=[
        pltpu.VMEM((_BW,), jnp.int32),
        pltpu.VMEM((_BW, 2 * _D), jnp.float32),
        pltpu.SemaphoreType.DMA,
    ],
)
def _gather_rows(feat_hbm, idx_hbm, out_hbm, idx_v, rows_v, sem):
    wid = lax.axis_index("s") * _NC + lax.axis_index("c")
    base = wid * _BW
    pltpu.sync_copy(idx_hbm.at[pl.ds(base, _BW)], idx_v)
    pltpu.async_copy(feat_hbm.at[idx_v], rows_v, sem).wait()
    pltpu.sync_copy(rows_v, out_hbm.at[pl.ds(base, _BW)])


def _combine_kernel(lse_ref, x_ref, rows_ref, par_ref, out_ref):
    row_t = jnp.where(par_ref[...] == 0, rows_ref[:, :_D], rows_ref[:, _D:])
    tgt = jnp.sum(x_ref[...] * row_t, axis=1, keepdims=True)
    out_ref[...] = jnp.sum(lse_ref[...] - tgt, keepdims=True) * (_LN2 / _B)


def kernel(inputs, features, targets):
    x = inputs * (_LOG2E / _TEMP)
    t = targets.astype(jnp.int32)

    lse2, feat_wide = pl.pallas_call(
        _lse_kernel,
        grid=(_GRID,),
        in_specs=[
            pl.BlockSpec((_B, _D), lambda i: (0, 0)),
            pl.BlockSpec((_BN, _D), lambda i: (i, 0)),
        ],
        out_specs=[
            pl.BlockSpec((_B, 1), lambda i: (0, 0)),
            pl.BlockSpec((_BN // 2, 2 * _D), lambda i: (i, 0)),
        ],
        out_shape=[
            jax.ShapeDtypeStruct((_B, 1), jnp.float32),
            jax.ShapeDtypeStruct((_N // 2, 2 * _D), jnp.float32),
        ],
        scratch_shapes=[
            pltpu.VMEM((_B, 1), jnp.float32),
            pltpu.VMEM((_B, 1), jnp.float32),
        ],
    )(x, features)

    rows = _gather_rows(feat_wide, t >> 1)
    parity = (t & 1).reshape(_B, 1)

    out = pl.pallas_call(
        _combine_kernel,
        out_shape=jax.ShapeDtypeStruct((1, 1), jnp.float32),
    )(lse2, x, rows, parity)
    return out[0, 0]


# speedup vs baseline: 1.1143x; 1.1143x over previous
import jax
import jax.numpy as jnp
from jax.experimental import pallas as pl
from jax.experimental.pallas import tpu as pltpu

_TEMP = 0.05
_B = 1024
_D = 64
_N = 100000
_BN = 4000
_GRID = _N // _BN
_LOG2E = 1.4426950408889634
_LN2 = 0.6931471805599453
_C2 = 84.0


def _ce_kernel(x_ref, f_ref, t_ref, out_ref, mc_ref, s_ref, g_ref):
    i = pl.program_id(0)

    @pl.when(i == 0)
    def _init():
        x2 = x_ref[...]
        m2 = jnp.sqrt(jnp.sum(x2 * x2, axis=1, keepdims=True))
        mc_ref[...] = m2 - _C2
        s_ref[...] = jnp.zeros_like(s_ref)
        g_ref[...] = jnp.zeros_like(g_ref)

    z = jax.lax.dot_general(
        x_ref[...].astype(jnp.bfloat16), f_ref[...],
        (((1,), (1,)), ((), ())),
        preferred_element_type=jnp.float32,
    )
    e = jnp.exp2(z - mc_ref[...])
    s_ref[...] += jnp.sum(e, axis=1, keepdims=True)

    col = jax.lax.broadcasted_iota(jnp.int32, z.shape, 1) + i * _BN
    hit = col == t_ref[...]
    g_ref[...] += jnp.sum(jnp.where(hit, z, 0.0), axis=1, keepdims=True)

    @pl.when(i == _GRID - 1)
    def _fin():
        lse2 = mc_ref[...] + jnp.log2(s_ref[...])
        out_ref[...] = jnp.sum(lse2 - g_ref[...], keepdims=True) * (_LN2 / _B)


def kernel(inputs, features, targets):
    x = inputs * (_LOG2E / _TEMP)
    fb = features.astype(jnp.bfloat16)
    t = targets.astype(jnp.int32).reshape(_B, 1)
    out = pl.pallas_call(
        _ce_kernel,
        grid=(_GRID,),
        in_specs=[
            pl.BlockSpec((_B, _D), lambda i: (0, 0)),
            pl.BlockSpec((_BN, _D), lambda i: (i, 0)),
            pl.BlockSpec((_B, 1), lambda i: (0, 0)),
        ],
        out_specs=pl.BlockSpec((1, 1), lambda i: (0, 0)),
        out_shape=jax.ShapeDtypeStruct((1, 1), jnp.float32),
        scratch_shapes=[
            pltpu.VMEM((_B, 1), jnp.float32),
            pltpu.VMEM((_B, 1), jnp.float32),
            pltpu.VMEM((_B, 1), jnp.float32),
        ],
    )(x, fb, t)
    return out[0, 0]


# speedup vs baseline: 1.1601x; 1.0411x over previous
import jax
import jax.numpy as jnp
from jax.experimental import pallas as pl
from jax.experimental.pallas import tpu as pltpu

_TEMP = 0.05
_B = 1024
_D = 64
_N = 100000
_BN = 10000
_GRID = _N // _BN
_LOG2E = 1.4426950408889634
_LN2 = 0.6931471805599453
_C2 = 84.0


def _ce_kernel(x_ref, f_ref, t_ref, out_ref, mc_ref, s_ref, g_ref):
    i = pl.program_id(0)

    @pl.when(i == 0)
    def _init():
        x2 = x_ref[...]
        m2 = jnp.sqrt(jnp.sum(x2 * x2, axis=1, keepdims=True))
        mc_ref[...] = m2 - _C2
        s_ref[...] = jnp.zeros_like(s_ref)
        g_ref[...] = jnp.zeros_like(g_ref)

    z = jax.lax.dot_general(
        x_ref[...].astype(jnp.bfloat16), f_ref[...],
        (((1,), (1,)), ((), ())),
        preferred_element_type=jnp.float32,
    )
    e = jnp.exp2(z - mc_ref[...])
    s_ref[...] += jnp.sum(e, axis=1, keepdims=True)

    col = jax.lax.broadcasted_iota(jnp.int32, z.shape, 1) + i * _BN
    hit = col == t_ref[...]
    g_ref[...] += jnp.sum(jnp.where(hit, z, 0.0), axis=1, keepdims=True)

    @pl.when(i == _GRID - 1)
    def _fin():
        lse2 = mc_ref[...] + jnp.log2(s_ref[...])
        out_ref[...] = jnp.sum(lse2 - g_ref[...], keepdims=True) * (_LN2 / _B)


def kernel(inputs, features, targets):
    x = inputs * (_LOG2E / _TEMP)
    fb = features.astype(jnp.bfloat16)
    t = targets.astype(jnp.int32).reshape(_B, 1)
    out = pl.pallas_call(
        _ce_kernel,
        grid=(_GRID,),
        in_specs=[
            pl.BlockSpec((_B, _D), lambda i: (0, 0)),
            pl.BlockSpec((_BN, _D), lambda i: (i, 0)),
            pl.BlockSpec((_B, 1), lambda i: (0, 0)),
        ],
        out_specs=pl.BlockSpec((1, 1), lambda i: (0, 0)),
        out_shape=jax.ShapeDtypeStruct((1, 1), jnp.float32),
        scratch_shapes=[
            pltpu.VMEM((_B, 1), jnp.float32),
            pltpu.VMEM((_B, 1), jnp.float32),
            pltpu.VMEM((_B, 1), jnp.float32),
        ],
    )(x, fb, t)
    return out[0, 0]
